# baseline (speedup 1.0000x reference)
import functools

import jax
import jax.numpy as jnp
from jax import lax
from jax.experimental import pallas as pl
from jax.experimental.pallas import tpu as pltpu

_MARGIN = 0.2
_S = 64.0
_BETA = 1.0
_GAMMA = 0.01


def _proser_block(costh_ref, label_ref, out_ref, *, bm, n_cols, n_blocks):
    i = pl.program_id(0)
    x = costh_ref[...]
    e = jnp.exp(x * _S - _S)

    col = lax.broadcasted_iota(jnp.int32, (bm, n_cols), 1)
    lab = label_ref[...]
    is_lab = col == lab

    ones_mat = jnp.ones((n_cols, 128), jnp.float32)
    e_full = lax.dot_general(
        e, ones_mat, (((1,), (0,)), ((), ())),
        precision=lax.Precision.HIGHEST,
    )[:, 0]
    x_masked = jnp.where(is_lab, x, 0.0)
    c = lax.dot_general(
        x_masked, ones_mat, (((1,), (0,)), ((), ())),
        precision=lax.Precision.HIGHEST,
    )[:, 0]
    e_oth = e_full - jnp.exp(c * _S - _S)
    last = x[:, n_cols - 1]

    cos_m = jnp.float32(jnp.cos(_MARGIN))
    sin_m = jnp.float32(jnp.sin(_MARGIN))
    v = _S * (c * cos_m - sin_m * jnp.sqrt(jnp.maximum(1.0 - c * c, 0.0)))

    lse1 = _S + jnp.log(e_oth + jnp.exp(v - _S))
    lse2 = _S + jnp.log(e_oth + jnp.exp(jnp.float32(-_S)))

    nll1 = lse1 - v
    t = jnp.where(lab[:, 0] == n_cols - 1, 0.0, _S * last)
    nll2 = lse2 - t

    first_half = i < (n_blocks // 2)
    w2 = jnp.where(first_half, _BETA, _GAMMA)
    contrib = (
        jnp.where(first_half, jnp.sum(nll1), 0.0) + w2 * jnp.sum(nll2)
    )

    @pl.when(i == 0)
    def _init():
        out_ref[0, 0] = 0.0

    out_ref[0, 0] += contrib


def kernel(costh, label, half_batch_size):
    B, C = costh.shape
    h = B // 2
    bm = 1024
    n_blocks = B // bm

    label2 = label.reshape(B, 1).astype(jnp.int32)

    total = pl.pallas_call(
        functools.partial(_proser_block, bm=bm, n_cols=C, n_blocks=n_blocks),
        grid=(n_blocks,),
        in_specs=[
            pl.BlockSpec((bm, C), lambda i: (i, 0)),
            pl.BlockSpec((bm, 1), lambda i: (i, 0)),
        ],
        out_specs=pl.BlockSpec(
            (1, 1), lambda i: (0, 0), memory_space=pltpu.SMEM
        ),
        out_shape=jax.ShapeDtypeStruct((1, 1), jnp.float32),
    )(costh, label2)

    return total[0, 0] / jnp.float32(h)


# speedup vs baseline: 1.6920x; 1.6920x over previous
import functools

import jax
import jax.numpy as jnp
from jax import lax
from jax.experimental import pallas as pl
from jax.experimental.pallas import tpu as pltpu

_MARGIN = 0.2
_S = 64.0
_BETA = 1.0
_GAMMA = 0.01


def _proser_block(costh_ref, label_ref, out_ref, *, bm, n_cols, n_blocks):
    i = pl.program_id(0)
    x = costh_ref[...]
    e = jnp.exp(x * _S - _S)

    col = lax.broadcasted_iota(jnp.int32, (bm, n_cols), 1)
    lab = label_ref[...]
    is_lab = col == lab

    ones_mat = jnp.ones((n_cols, 128), jnp.float32)
    e_full = lax.dot_general(
        e, ones_mat, (((1,), (0,)), ((), ())),
        precision=lax.Precision.DEFAULT,
    )[:, 0]
    x_masked = jnp.where(is_lab, x, 0.0)
    c = lax.dot_general(
        x_masked, ones_mat, (((1,), (0,)), ((), ())),
        precision=lax.Precision.DEFAULT,
    )[:, 0]
    e_oth = e_full - jnp.exp(c * _S - _S)
    last = x[:, n_cols - 1]

    cos_m = jnp.float32(jnp.cos(_MARGIN))
    sin_m = jnp.float32(jnp.sin(_MARGIN))
    v = _S * (c * cos_m - sin_m * jnp.sqrt(jnp.maximum(1.0 - c * c, 0.0)))

    lse1 = _S + jnp.log(e_oth + jnp.exp(v - _S))
    lse2 = _S + jnp.log(e_oth + jnp.exp(jnp.float32(-_S)))

    nll1 = lse1 - v
    t = jnp.where(lab[:, 0] == n_cols - 1, 0.0, _S * last)
    nll2 = lse2 - t

    first_half = i < (n_blocks // 2)
    w2 = jnp.where(first_half, _BETA, _GAMMA)
    contrib = (
        jnp.where(first_half, jnp.sum(nll1), 0.0) + w2 * jnp.sum(nll2)
    )

    @pl.when(i == 0)
    def _init():
        out_ref[0, 0] = 0.0

    out_ref[0, 0] += contrib


def kernel(costh, label, half_batch_size):
    B, C = costh.shape
    h = B // 2
    bm = 1024
    n_blocks = B // bm

    label2 = label.reshape(B, 1).astype(jnp.int32)

    total = pl.pallas_call(
        functools.partial(_proser_block, bm=bm, n_cols=C, n_blocks=n_blocks),
        grid=(n_blocks,),
        in_specs=[
            pl.BlockSpec((bm, C), lambda i: (i, 0)),
            pl.BlockSpec((bm, 1), lambda i: (i, 0)),
        ],
        out_specs=pl.BlockSpec(
            (1, 1), lambda i: (0, 0), memory_space=pltpu.SMEM
        ),
        out_shape=jax.ShapeDtypeStruct((1, 1), jnp.float32),
    )(costh, label2)

    return total[0, 0] / jnp.float32(h)


# speedup vs baseline: 1.7460x; 1.0319x over previous
import functools

import jax
import jax.numpy as jnp
from jax import lax
from jax.experimental import pallas as pl
from jax.experimental.pallas import tpu as pltpu

_MARGIN = 0.2
_S = 64.0
_BETA = 1.0
_GAMMA = 0.01
_NSTREAM = 4


def _stream_contrib(x, lab, bm, n_cols, first_half):
    e = jnp.exp(x * _S - _S)

    col = lax.broadcasted_iota(jnp.int32, (bm, n_cols), 1)
    is_lab = col == lab

    ones_mat = jnp.ones((n_cols, 128), jnp.float32)
    e_full = lax.dot_general(
        e, ones_mat, (((1,), (0,)), ((), ()))
    )[:, 0]
    c = lax.dot_general(
        jnp.where(is_lab, x, 0.0), ones_mat, (((1,), (0,)), ((), ()))
    )[:, 0]
    e_oth = e_full - jnp.exp(c * _S - _S)
    last = x[:, n_cols - 1]

    cos_m = jnp.float32(jnp.cos(_MARGIN))
    sin_m = jnp.float32(jnp.sin(_MARGIN))
    v = _S * (c * cos_m - sin_m * jnp.sqrt(jnp.maximum(1.0 - c * c, 0.0)))

    lse2 = _S + jnp.log(e_oth + jnp.exp(jnp.float32(-_S)))
    t = jnp.where(lab[:, 0] == n_cols - 1, 0.0, _S * last)
    nll2 = lse2 - t

    if first_half:
        lse1 = _S + jnp.log(e_oth + jnp.exp(v - _S))
        nll1 = lse1 - v
        return jnp.sum(nll1) + _BETA * jnp.sum(nll2)
    return _GAMMA * jnp.sum(nll2)


def _proser_block(*refs, bm, n_cols):
    costh_refs = refs[:_NSTREAM]
    label_refs = refs[_NSTREAM:2 * _NSTREAM]
    out_ref = refs[2 * _NSTREAM]
    i = pl.program_id(0)

    contrib = jnp.float32(0.0)
    for s in range(_NSTREAM):
        contrib += _stream_contrib(
            costh_refs[s][...],
            label_refs[s][...],
            bm,
            n_cols,
            first_half=(s < _NSTREAM // 2),
        )

    @pl.when(i == 0)
    def _init():
        out_ref[0, 0] = 0.0

    out_ref[0, 0] += contrib


def kernel(costh, label, half_batch_size):
    B, C = costh.shape
    h = B // 2
    bm = 256
    n_blocks = (B // _NSTREAM) // bm

    label2 = label.reshape(B, 1).astype(jnp.int32)

    costh_specs = [
        pl.BlockSpec((bm, C), lambda i, q=q, nb=n_blocks: (i + q * nb, 0))
        for q in range(_NSTREAM)
    ]
    label_specs = [
        pl.BlockSpec((bm, 1), lambda i, q=q, nb=n_blocks: (i + q * nb, 0))
        for q in range(_NSTREAM)
    ]

    total = pl.pallas_call(
        functools.partial(_proser_block, bm=bm, n_cols=C),
        grid=(n_blocks,),
        in_specs=costh_specs + label_specs,
        out_specs=pl.BlockSpec(
            (1, 1), lambda i: (0, 0), memory_space=pltpu.SMEM
        ),
        out_shape=jax.ShapeDtypeStruct((1, 1), jnp.float32),
    )(*([costh] * _NSTREAM), *([label2] * _NSTREAM))

    return total[0, 0] / jnp.float32(h)
